# Initial kernel scaffold; baseline (speedup 1.0000x reference)
import jax
import jax.numpy as jnp
from jax.experimental import pallas as pl


def kernel(protein_h, attn_w, proj_w, proj_b, drug_idx, protein_indices, pathway_segment_ids, pathway_to_drug):
    raise NotImplementedError("write your pallas kernel here")


# speedup vs baseline: 9.5561x; 9.5561x over previous
import functools

import jax
import jax.numpy as jnp
from jax import lax
from jax.experimental import pallas as pl
from jax.experimental.pallas import tpu as pltpu
from jax.experimental.pallas import tpu_sc as plsc

NC = 2
NS = 16
NW = NC * NS
L = 16
K = 80


def _escore_tc(protein_h, attn_w):
    def body(ph_ref, aw_ref, out_ref):
        s = jnp.dot(ph_ref[...], aw_ref[...], preferred_element_type=jnp.float32)
        out_ref[...] = jnp.exp(s - jnp.max(s))
    return pl.pallas_call(
        body,
        out_shape=jax.ShapeDtypeStruct((protein_h.shape[0], 1), jnp.float32),
    )(protein_h, attn_w)


def _denom_sc(t_idx, seg_ids, e_all, zeros_p, n_pathways):
    m = t_idx.shape[0]
    per_w = m // NW
    nch = per_w // K
    mesh = plsc.VectorSubcoreMesh(core_axis_name="c", subcore_axis_name="s",
                                  num_cores=NC, num_subcores=NS)

    @functools.partial(
        pl.kernel,
        out_type=jax.ShapeDtypeStruct((NC, n_pathways), jnp.float32),
        mesh=mesh,
        compiler_params=pltpu.CompilerParams(needs_layout_passes=False),
        scratch_types=[
            pltpu.VMEM((e_all.shape[0],), jnp.float32),
            pltpu.VMEM((K,), jnp.int32),
            pltpu.VMEM((1, K), jnp.int32),
            pltpu.VMEM((1, K), jnp.float32),
            pltpu.VMEM_SHARED((n_pathways,), jnp.float32),
        ],
    )
    def kern(t_hbm, seg_hbm, e_hbm, z_hbm, out_hbm, e_tab, t_buf, s_buf, e_buf, den_sh):
        c = lax.axis_index("c")
        s = lax.axis_index("s")
        wid = s * NC + c
        pltpu.sync_copy(e_hbm, e_tab)

        @pl.when(s == 0)
        def _():
            pltpu.sync_copy(z_hbm, den_sh)

        plsc.subcore_barrier()
        base = wid * per_w

        def body(g, carry):
            off = base + g * K
            pltpu.sync_copy(t_hbm.at[pl.ds(off, K)], t_buf)
            pltpu.sync_copy(seg_hbm.at[pl.ds(off, K)], s_buf.at[0])
            for j in range(K // L):
                sl = pl.ds(j * L, L)
                e_buf[0, sl] = plsc.load_gather(e_tab, [t_buf[sl]])
            pltpu.sync_copy(e_buf.at[0], den_sh.at[s_buf.at[0]], add=True)
            return carry

        lax.fori_loop(0, nch, body, 0)
        plsc.subcore_barrier()

        @pl.when(s == 0)
        def _():
            pltpu.sync_copy(den_sh, out_hbm.at[c])

    return kern(t_idx, seg_ids, e_all, zeros_p)


def _pool_sc(protein_h, t_idx, seg_ids, e_all, denom, ptd, zeros_nd, zeros_n, n_drugs):
    m = t_idx.shape[0]
    d = protein_h.shape[1]
    per_w = m // NW
    nch = per_w // K
    n_pathways = ptd.shape[0]
    mesh = plsc.VectorSubcoreMesh(core_axis_name="c", subcore_axis_name="s",
                                  num_cores=NC, num_subcores=NS)

    @functools.partial(
        pl.kernel,
        out_type=(
            jax.ShapeDtypeStruct((NC, n_drugs, d), jnp.float32),
            jax.ShapeDtypeStruct((NC, n_drugs), jnp.float32),
        ),
        mesh=mesh,
        compiler_params=pltpu.CompilerParams(needs_layout_passes=False),
        scratch_types=[
            pltpu.VMEM((e_all.shape[0],), jnp.float32),
            pltpu.VMEM((K,), jnp.int32),
            pltpu.VMEM((K,), jnp.int32),
            pltpu.VMEM((K,), jnp.float32),
            pltpu.VMEM((1, K), jnp.int32),
            pltpu.VMEM((1, K), jnp.float32),
            pltpu.VMEM((K, d), jnp.float32),
            pltpu.SemaphoreType.DMA,
            pltpu.SemaphoreType.DMA,
            pltpu.SemaphoreType.DMA,
            pltpu.VMEM_SHARED((n_drugs, d), jnp.float32),
            pltpu.VMEM_SHARED((n_drugs,), jnp.float32),
        ],
    )
    def kern(ph_hbm, t_hbm, seg_hbm, e_hbm, den_hbm, ptd_hbm, znd_hbm, zn_hbm,
             dsum_hbm, cnt_hbm,
             e_tab, t_buf, s_buf, dn_buf, dg_buf, cf_buf, row_buf,
             sem, sem2, sem3, dsum_sh, cnt_sh):
        c = lax.axis_index("c")
        s = lax.axis_index("s")
        wid = s * NC + c
        pltpu.sync_copy(e_hbm, e_tab)

        @pl.when(s == 0)
        def _():
            pltpu.sync_copy(znd_hbm, dsum_sh)
            pltpu.sync_copy(zn_hbm, cnt_sh)

        plsc.subcore_barrier()
        base = wid * per_w

        def body(g, carry):
            off = base + g * K
            pltpu.sync_copy(t_hbm.at[pl.ds(off, K)], t_buf)
            pltpu.sync_copy(seg_hbm.at[pl.ds(off, K)], s_buf)
            gather = pltpu.async_copy(ph_hbm.at[t_buf], row_buf, sem)
            g_den = pltpu.async_copy(den_hbm.at[s_buf], dn_buf, sem2)
            g_ptd = pltpu.async_copy(ptd_hbm.at[s_buf], dg_buf.at[0], sem3)
            g_den.wait()
            g_ptd.wait()
            for j in range(K // L):
                sl = pl.ds(j * L, L)
                e16 = plsc.load_gather(e_tab, [t_buf[sl]])
                cf_buf[0, sl] = e16 / dn_buf[sl]
            gather.wait()

            def rbody(r, rc):
                csp = plsc.load_gather(
                    cf_buf, [jnp.zeros((L,), jnp.int32), jnp.full((L,), r, jnp.int32)])
                for q in range(d // L):
                    sl2 = pl.ds(q * L, L)
                    row_buf[r, sl2] = row_buf[r, sl2] * csp
                return rc

            lax.fori_loop(0, K, rbody, 0)
            pltpu.sync_copy(row_buf, dsum_sh.at[dg_buf.at[0]], add=True)
            pltpu.sync_copy(cf_buf.at[0], cnt_sh.at[dg_buf.at[0]], add=True)
            return carry

        lax.fori_loop(0, nch, body, 0)
        plsc.subcore_barrier()

        @pl.when(s == 0)
        def _():
            pltpu.sync_copy(dsum_sh, dsum_hbm.at[c])
            pltpu.sync_copy(cnt_sh, cnt_hbm.at[c])

    return kern(protein_h, t_idx, seg_ids, e_all, denom, ptd, zeros_nd, zeros_n)


def _finish_tc(dsum2, cnt2, proj_wT, proj_b):
    n, d = dsum2.shape[1], dsum2.shape[2]

    def body(ds_ref, ct_ref, pw_ref, pb_ref, out_ref):
        tot = ds_ref[0] + ds_ref[1]
        cnt = ct_ref[0] + ct_ref[1]
        avg = tot / jnp.maximum(cnt, 1.0)[:, None]
        r = jnp.dot(avg, pw_ref[...], preferred_element_type=jnp.float32)
        out_ref[...] = jnp.maximum(r + pb_ref[...], 0.0)

    return pl.pallas_call(
        body,
        out_shape=jax.ShapeDtypeStruct((n, d), jnp.float32),
    )(dsum2, cnt2, proj_wT, proj_b)


def kernel(protein_h, attn_w, proj_w, proj_b, drug_idx, protein_indices,
           pathway_segment_ids, pathway_to_drug):
    n_drugs = drug_idx.shape[0]
    n_pathways = pathway_to_drug.shape[0]
    d = protein_h.shape[1]

    e_all = _escore_tc(protein_h, attn_w)[:, 0]
    zeros_p = jnp.zeros((n_pathways,), jnp.float32)
    den2 = _denom_sc(protein_indices, pathway_segment_ids, e_all, zeros_p,
                     n_pathways)
    denom = den2[0] + den2[1]
    zeros_nd = jnp.zeros((n_drugs, d), jnp.float32)
    zeros_n = jnp.zeros((n_drugs,), jnp.float32)
    dsum2, cnt2 = _pool_sc(protein_h, protein_indices, pathway_segment_ids,
                           e_all, denom, pathway_to_drug, zeros_nd, zeros_n,
                           n_drugs)
    return _finish_tc(dsum2, cnt2, proj_w.T, jnp.reshape(proj_b, (1, d)))


# speedup vs baseline: 12.2664x; 1.2836x over previous
import functools

import jax
import jax.numpy as jnp
from jax import lax
from jax.experimental import pallas as pl
from jax.experimental.pallas import tpu as pltpu
from jax.experimental.pallas import tpu_sc as plsc

NC = 2
NS = 16
NW = NC * NS
L = 16
K = 80


def _escore_tc(protein_h, attn_w):
    def body(ph_ref, aw_ref, out_ref):
        s = jnp.dot(ph_ref[...], aw_ref[...], preferred_element_type=jnp.float32)
        out_ref[...] = jnp.exp(s - jnp.max(s))
    return pl.pallas_call(
        body,
        out_shape=jax.ShapeDtypeStruct((protein_h.shape[0], 1), jnp.float32),
    )(protein_h, attn_w)


def _denom_sc(t_idx, seg_ids, e_all, zeros_p, n_pathways):
    m = t_idx.shape[0]
    per_w = m // NW
    nch = per_w // K
    mesh = plsc.VectorSubcoreMesh(core_axis_name="c", subcore_axis_name="s",
                                  num_cores=NC, num_subcores=NS)

    @functools.partial(
        pl.kernel,
        out_type=jax.ShapeDtypeStruct((NC, n_pathways), jnp.float32),
        mesh=mesh,
        compiler_params=pltpu.CompilerParams(needs_layout_passes=False),
        scratch_types=[
            pltpu.VMEM((e_all.shape[0],), jnp.float32),
            pltpu.VMEM((K,), jnp.int32),
            pltpu.VMEM((1, K), jnp.int32),
            pltpu.VMEM((1, K), jnp.float32),
            pltpu.VMEM_SHARED((n_pathways,), jnp.float32),
        ],
    )
    def kern(t_hbm, seg_hbm, e_hbm, z_hbm, out_hbm, e_tab, t_buf, s_buf, e_buf, den_sh):
        c = lax.axis_index("c")
        s = lax.axis_index("s")
        wid = s * NC + c
        pltpu.sync_copy(e_hbm, e_tab)

        @pl.when(s == 0)
        def _():
            pltpu.sync_copy(z_hbm, den_sh)

        plsc.subcore_barrier()
        base = wid * per_w

        def body(g, carry):
            off = base + g * K
            pltpu.sync_copy(t_hbm.at[pl.ds(off, K)], t_buf)
            pltpu.sync_copy(seg_hbm.at[pl.ds(off, K)], s_buf.at[0])
            for j in range(K // L):
                sl = pl.ds(j * L, L)
                e_buf[0, sl] = plsc.load_gather(e_tab, [t_buf[sl]])
            pltpu.sync_copy(e_buf.at[0], den_sh.at[s_buf.at[0]], add=True)
            return carry

        lax.fori_loop(0, nch, body, 0)
        plsc.subcore_barrier()

        @pl.when(s == 0)
        def _():
            pltpu.sync_copy(den_sh, out_hbm.at[c])

    return kern(t_idx, seg_ids, e_all, zeros_p)


def _pool_sc(protein_h, t_idx, seg_ids, e_all, denom, ptd, zeros_nd, zeros_n, n_drugs):
    m = t_idx.shape[0]
    d = protein_h.shape[1]
    per_w = m // NW
    nch = per_w // K
    n_pathways = ptd.shape[0]
    mesh = plsc.VectorSubcoreMesh(core_axis_name="c", subcore_axis_name="s",
                                  num_cores=NC, num_subcores=NS)

    @functools.partial(
        pl.kernel,
        out_type=(
            jax.ShapeDtypeStruct((NC, n_drugs, d), jnp.float32),
            jax.ShapeDtypeStruct((NC, n_drugs), jnp.float32),
        ),
        mesh=mesh,
        compiler_params=pltpu.CompilerParams(needs_layout_passes=False),
        scratch_types=[
            pltpu.VMEM((e_all.shape[0],), jnp.float32),
            pltpu.VMEM((2, K), jnp.int32),
            pltpu.VMEM((2, K), jnp.int32),
            pltpu.VMEM((2, K), jnp.float32),
            pltpu.VMEM((2, K), jnp.int32),
            pltpu.VMEM((2, K), jnp.float32),
            pltpu.VMEM((2, K, d), jnp.float32),
            pltpu.SemaphoreType.DMA,
            pltpu.SemaphoreType.DMA,
            pltpu.SemaphoreType.DMA,
            pltpu.SemaphoreType.DMA,
            pltpu.SemaphoreType.DMA,
            pltpu.SemaphoreType.DMA,
            pltpu.VMEM_SHARED((n_drugs, d), jnp.float32),
            pltpu.VMEM_SHARED((n_drugs,), jnp.float32),
        ],
    )
    def kern(ph_hbm, t_hbm, seg_hbm, e_hbm, den_hbm, ptd_hbm, znd_hbm, zn_hbm,
             dsum_hbm, cnt_hbm,
             e_tab, t_buf, s_buf, dn_buf, dg_buf, cf_buf, row_buf,
             semr0, semr1, semd0, semd1, semp0, semp1, dsum_sh, cnt_sh):
        c = lax.axis_index("c")
        s = lax.axis_index("s")
        wid = s * NC + c
        pltpu.sync_copy(e_hbm, e_tab)

        @pl.when(s == 0)
        def _():
            pltpu.sync_copy(znd_hbm, dsum_sh)
            pltpu.sync_copy(zn_hbm, cnt_sh)

        plsc.subcore_barrier()
        base = wid * per_w
        semr = (semr0, semr1)
        semd = (semd0, semd1)
        semp = (semp0, semp1)

        def issue(g, b):
            off = base + g * K
            pltpu.sync_copy(t_hbm.at[pl.ds(off, K)], t_buf.at[b])
            pltpu.sync_copy(seg_hbm.at[pl.ds(off, K)], s_buf.at[b])
            pltpu.async_copy(ph_hbm.at[t_buf.at[b]], row_buf.at[b], semr[b])
            pltpu.async_copy(den_hbm.at[s_buf.at[b]], dn_buf.at[b], semd[b])
            pltpu.async_copy(ptd_hbm.at[s_buf.at[b]], dg_buf.at[b], semp[b])

        def process(b):
            pltpu.make_async_copy(den_hbm.at[s_buf.at[b]], dn_buf.at[b],
                                  semd[b]).wait()
            pltpu.make_async_copy(ptd_hbm.at[s_buf.at[b]], dg_buf.at[b],
                                  semp[b]).wait()
            for j in range(K // L):
                sl = pl.ds(j * L, L)
                e16 = plsc.load_gather(e_tab, [t_buf[b, sl]])
                cf_buf[b, sl] = e16 / dn_buf[b, sl]
            pltpu.make_async_copy(ph_hbm.at[t_buf.at[b]], row_buf.at[b],
                                  semr[b]).wait()

            def rbody(r, rc):
                csp = plsc.load_gather(
                    cf_buf,
                    [jnp.full((L,), b, jnp.int32), jnp.full((L,), r, jnp.int32)])
                for q in range(d // L):
                    sl2 = pl.ds(q * L, L)
                    row_buf[b, r, sl2] = row_buf[b, r, sl2] * csp
                return rc

            lax.fori_loop(0, K, rbody, 0)
            pltpu.sync_copy(row_buf.at[b], dsum_sh.at[dg_buf.at[b]], add=True)
            pltpu.sync_copy(cf_buf.at[b], cnt_sh.at[dg_buf.at[b]], add=True)

        issue(0, 0)

        def pair(i, carry):
            issue(2 * i + 1, 1)
            process(0)
            issue(2 * i + 2, 0)
            process(1)
            return carry

        lax.fori_loop(0, (nch - 1) // 2, pair, 0)
        process(0)
        plsc.subcore_barrier()

        @pl.when(s == 0)
        def _():
            pltpu.sync_copy(dsum_sh, dsum_hbm.at[c])
            pltpu.sync_copy(cnt_sh, cnt_hbm.at[c])

    return kern(protein_h, t_idx, seg_ids, e_all, denom, ptd, zeros_nd, zeros_n)


def _finish_tc(dsum2, cnt2, proj_wT, proj_b):
    n, d = dsum2.shape[1], dsum2.shape[2]

    def body(ds_ref, ct_ref, pw_ref, pb_ref, out_ref):
        tot = ds_ref[0] + ds_ref[1]
        cnt = ct_ref[0] + ct_ref[1]
        avg = tot / jnp.maximum(cnt, 1.0)[:, None]
        r = jnp.dot(avg, pw_ref[...], preferred_element_type=jnp.float32)
        out_ref[...] = jnp.maximum(r + pb_ref[...], 0.0)

    return pl.pallas_call(
        body,
        out_shape=jax.ShapeDtypeStruct((n, d), jnp.float32),
    )(dsum2, cnt2, proj_wT, proj_b)


def kernel(protein_h, attn_w, proj_w, proj_b, drug_idx, protein_indices,
           pathway_segment_ids, pathway_to_drug):
    n_drugs = drug_idx.shape[0]
    n_pathways = pathway_to_drug.shape[0]
    d = protein_h.shape[1]

    e_all = _escore_tc(protein_h, attn_w)[:, 0]
    zeros_p = jnp.zeros((n_pathways,), jnp.float32)
    den2 = _denom_sc(protein_indices, pathway_segment_ids, e_all, zeros_p,
                     n_pathways)
    denom = den2[0] + den2[1]
    zeros_nd = jnp.zeros((n_drugs, d), jnp.float32)
    zeros_n = jnp.zeros((n_drugs,), jnp.float32)
    dsum2, cnt2 = _pool_sc(protein_h, protein_indices, pathway_segment_ids,
                           e_all, denom, pathway_to_drug, zeros_nd, zeros_n,
                           n_drugs)
    return _finish_tc(dsum2, cnt2, proj_w.T, jnp.reshape(proj_b, (1, d)))


# speedup vs baseline: 12.6663x; 1.0326x over previous
import functools

import jax
import jax.numpy as jnp
from jax import lax
from jax.experimental import pallas as pl
from jax.experimental.pallas import tpu as pltpu
from jax.experimental.pallas import tpu_sc as plsc

NC = 2
NS = 16
NW = NC * NS
L = 16
K = 80


def _escore_tc(protein_h, attn_w):
    def body(ph_ref, aw_ref, out_ref):
        s = jnp.dot(ph_ref[...], aw_ref[...], preferred_element_type=jnp.float32)
        out_ref[...] = jnp.exp(s - jnp.max(s))
    return pl.pallas_call(
        body,
        out_shape=jax.ShapeDtypeStruct((protein_h.shape[0], 1), jnp.float32),
    )(protein_h, attn_w)


def _denom_sc(t_idx, seg_ids, e_all, zeros_p, n_pathways):
    m = t_idx.shape[0]
    per_w = m // NW
    nch = per_w // K
    mesh = plsc.VectorSubcoreMesh(core_axis_name="c", subcore_axis_name="s",
                                  num_cores=NC, num_subcores=NS)

    @functools.partial(
        pl.kernel,
        out_type=jax.ShapeDtypeStruct((NC, n_pathways), jnp.float32),
        mesh=mesh,
        compiler_params=pltpu.CompilerParams(needs_layout_passes=False),
        scratch_types=[
            pltpu.VMEM((e_all.shape[0],), jnp.float32),
            pltpu.VMEM((K,), jnp.int32),
            pltpu.VMEM((1, K), jnp.int32),
            pltpu.VMEM((1, K), jnp.float32),
            pltpu.VMEM_SHARED((n_pathways,), jnp.float32),
        ],
    )
    def kern(t_hbm, seg_hbm, e_hbm, z_hbm, out_hbm, e_tab, t_buf, s_buf, e_buf, den_sh):
        c = lax.axis_index("c")
        s = lax.axis_index("s")
        wid = s * NC + c
        pltpu.sync_copy(e_hbm, e_tab)

        @pl.when(s == 0)
        def _():
            pltpu.sync_copy(z_hbm, den_sh)

        plsc.subcore_barrier()
        base = wid * per_w

        def body(g, carry):
            off = base + g * K
            pltpu.sync_copy(t_hbm.at[pl.ds(off, K)], t_buf)
            pltpu.sync_copy(seg_hbm.at[pl.ds(off, K)], s_buf.at[0])
            for j in range(K // L):
                sl = pl.ds(j * L, L)
                e_buf[0, sl] = plsc.load_gather(e_tab, [t_buf[sl]])
            pltpu.sync_copy(e_buf.at[0], den_sh.at[s_buf.at[0]], add=True)
            return carry

        lax.fori_loop(0, nch, body, 0)
        plsc.subcore_barrier()

        @pl.when(s == 0)
        def _():
            pltpu.sync_copy(den_sh, out_hbm.at[c])

    return kern(t_idx, seg_ids, e_all, zeros_p)


def _pool_sc(protein_h, t_idx, seg_ids, e_all, denom, ptd, zeros_nd, zeros_n, n_drugs):
    m = t_idx.shape[0]
    d = protein_h.shape[1]
    per_w = m // NW
    nch = per_w // K
    n_pathways = ptd.shape[0]
    mesh = plsc.VectorSubcoreMesh(core_axis_name="c", subcore_axis_name="s",
                                  num_cores=NC, num_subcores=NS)

    @functools.partial(
        pl.kernel,
        out_type=(
            jax.ShapeDtypeStruct((NC, n_drugs, d), jnp.float32),
            jax.ShapeDtypeStruct((NC, n_drugs), jnp.float32),
        ),
        mesh=mesh,
        compiler_params=pltpu.CompilerParams(needs_layout_passes=False),
        scratch_types=[
            pltpu.VMEM((e_all.shape[0],), jnp.float32),
            pltpu.VMEM((2, K), jnp.int32),
            pltpu.VMEM((2, K), jnp.int32),
            pltpu.VMEM((2, K), jnp.float32),
            pltpu.VMEM((2, K), jnp.int32),
            pltpu.VMEM((2, K), jnp.float32),
            pltpu.VMEM((2, K, d), jnp.float32),
            pltpu.SemaphoreType.DMA,
            pltpu.SemaphoreType.DMA,
            pltpu.SemaphoreType.DMA,
            pltpu.SemaphoreType.DMA,
            pltpu.SemaphoreType.DMA,
            pltpu.SemaphoreType.DMA,
            pltpu.SemaphoreType.DMA,
            pltpu.SemaphoreType.DMA,
            pltpu.SemaphoreType.DMA,
            pltpu.SemaphoreType.DMA,
            pltpu.VMEM_SHARED((n_drugs, d), jnp.float32),
            pltpu.VMEM_SHARED((n_drugs,), jnp.float32),
        ],
    )
    def kern(ph_hbm, t_hbm, seg_hbm, e_hbm, den_hbm, ptd_hbm, znd_hbm, zn_hbm,
             dsum_hbm, cnt_hbm,
             e_tab, t_buf, s_buf, dn_buf, dg_buf, cf_buf, row_buf,
             semr0, semr1, semd0, semd1, semp0, semp1,
             semw0, semw1, semc0, semc1, dsum_sh, cnt_sh):
        c = lax.axis_index("c")
        s = lax.axis_index("s")
        wid = s * NC + c
        pltpu.sync_copy(e_hbm, e_tab)

        @pl.when(s == 0)
        def _():
            pltpu.sync_copy(znd_hbm, dsum_sh)
            pltpu.sync_copy(zn_hbm, cnt_sh)

        plsc.subcore_barrier()
        base = wid * per_w
        semr = (semr0, semr1)
        semd = (semd0, semd1)
        semp = (semp0, semp1)
        semw = (semw0, semw1)
        semc = (semc0, semc1)

        def wait_scatter(b):
            pltpu.make_async_copy(row_buf.at[b], dsum_sh.at[dg_buf.at[b]],
                                  semw[b]).wait()
            pltpu.make_async_copy(cf_buf.at[b], cnt_sh.at[dg_buf.at[b]],
                                  semc[b]).wait()

        def issue(g, b, first=False):
            if not first:
                wait_scatter(b)
            off = base + g * K
            pltpu.sync_copy(t_hbm.at[pl.ds(off, K)], t_buf.at[b])
            pltpu.sync_copy(seg_hbm.at[pl.ds(off, K)], s_buf.at[b])
            pltpu.async_copy(ph_hbm.at[t_buf.at[b]], row_buf.at[b], semr[b])
            pltpu.async_copy(den_hbm.at[s_buf.at[b]], dn_buf.at[b], semd[b])
            pltpu.async_copy(ptd_hbm.at[s_buf.at[b]], dg_buf.at[b], semp[b])

        def process(b):
            pltpu.make_async_copy(den_hbm.at[s_buf.at[b]], dn_buf.at[b],
                                  semd[b]).wait()
            pltpu.make_async_copy(ptd_hbm.at[s_buf.at[b]], dg_buf.at[b],
                                  semp[b]).wait()
            for j in range(K // L):
                sl = pl.ds(j * L, L)
                e16 = plsc.load_gather(e_tab, [t_buf[b, sl]])
                cf_buf[b, sl] = e16 / dn_buf[b, sl]
            pltpu.make_async_copy(ph_hbm.at[t_buf.at[b]], row_buf.at[b],
                                  semr[b]).wait()

            def rbody(i, rc):
                for u in range(2):
                    r = 2 * i + u
                    csp = plsc.load_gather(
                        cf_buf,
                        [jnp.full((L,), b, jnp.int32),
                         jnp.full((L,), r, jnp.int32)])
                    for q in range(d // L):
                        sl2 = pl.ds(q * L, L)
                        row_buf[b, r, sl2] = row_buf[b, r, sl2] * csp
                return rc

            lax.fori_loop(0, K // 2, rbody, 0)
            pltpu.make_async_copy(row_buf.at[b], dsum_sh.at[dg_buf.at[b]],
                                  semw[b]).start(add=True)
            pltpu.make_async_copy(cf_buf.at[b], cnt_sh.at[dg_buf.at[b]],
                                  semc[b]).start(add=True)

        issue(0, 0, first=True)
        issue(1, 1, first=True)
        process(0)
        issue(2, 0)
        process(1)

        def pair(i, carry):
            issue(2 * i + 1, 1)
            process(0)
            issue(2 * i + 2, 0)
            process(1)
            return carry

        lax.fori_loop(1, (nch - 1) // 2, pair, 0)
        process(0)
        wait_scatter(1)
        wait_scatter(0)
        plsc.subcore_barrier()

        @pl.when(s == 0)
        def _():
            pltpu.sync_copy(dsum_sh, dsum_hbm.at[c])
            pltpu.sync_copy(cnt_sh, cnt_hbm.at[c])

    return kern(protein_h, t_idx, seg_ids, e_all, denom, ptd, zeros_nd, zeros_n)


def _finish_tc(dsum2, cnt2, proj_wT, proj_b):
    n, d = dsum2.shape[1], dsum2.shape[2]

    def body(ds_ref, ct_ref, pw_ref, pb_ref, out_ref):
        tot = ds_ref[0] + ds_ref[1]
        cnt = ct_ref[0] + ct_ref[1]
        avg = tot / jnp.maximum(cnt, 1.0)[:, None]
        r = jnp.dot(avg, pw_ref[...], preferred_element_type=jnp.float32)
        out_ref[...] = jnp.maximum(r + pb_ref[...], 0.0)

    return pl.pallas_call(
        body,
        out_shape=jax.ShapeDtypeStruct((n, d), jnp.float32),
    )(dsum2, cnt2, proj_wT, proj_b)


def kernel(protein_h, attn_w, proj_w, proj_b, drug_idx, protein_indices,
           pathway_segment_ids, pathway_to_drug):
    n_drugs = drug_idx.shape[0]
    n_pathways = pathway_to_drug.shape[0]
    d = protein_h.shape[1]

    e_all = _escore_tc(protein_h, attn_w)[:, 0]
    zeros_p = jnp.zeros((n_pathways,), jnp.float32)
    den2 = _denom_sc(protein_indices, pathway_segment_ids, e_all, zeros_p,
                     n_pathways)
    denom = den2[0] + den2[1]
    zeros_nd = jnp.zeros((n_drugs, d), jnp.float32)
    zeros_n = jnp.zeros((n_drugs,), jnp.float32)
    dsum2, cnt2 = _pool_sc(protein_h, protein_indices, pathway_segment_ids,
                           e_all, denom, pathway_to_drug, zeros_nd, zeros_n,
                           n_drugs)
    return _finish_tc(dsum2, cnt2, proj_w.T, jnp.reshape(proj_b, (1, d)))


# speedup vs baseline: 12.8469x; 1.0143x over previous
import functools

import jax
import jax.numpy as jnp
from jax import lax
from jax.experimental import pallas as pl
from jax.experimental.pallas import tpu as pltpu
from jax.experimental.pallas import tpu_sc as plsc

NC = 2
NS = 16
NW = NC * NS
L = 16
K = 80


def _escore_tc(protein_h, attn_w):
    def body(ph_ref, aw_ref, out_ref):
        s = jnp.dot(ph_ref[...], aw_ref[...], preferred_element_type=jnp.float32)
        out_ref[...] = jnp.exp(s - jnp.max(s))
    return pl.pallas_call(
        body,
        out_shape=jax.ShapeDtypeStruct((protein_h.shape[0], 1), jnp.float32),
    )(protein_h, attn_w)


def _denom_sc(t_idx, seg_ids, e_all, zeros_p, n_pathways):
    m = t_idx.shape[0]
    per_w = m // NW
    nch = per_w // K
    mesh = plsc.VectorSubcoreMesh(core_axis_name="c", subcore_axis_name="s",
                                  num_cores=NC, num_subcores=NS)

    @functools.partial(
        pl.kernel,
        out_type=jax.ShapeDtypeStruct((NC, n_pathways), jnp.float32),
        mesh=mesh,
        compiler_params=pltpu.CompilerParams(needs_layout_passes=False),
        scratch_types=[
            pltpu.VMEM((e_all.shape[0],), jnp.float32),
            pltpu.VMEM((K,), jnp.int32),
            pltpu.VMEM((1, K), jnp.int32),
            pltpu.VMEM((1, K), jnp.float32),
            pltpu.VMEM_SHARED((n_pathways,), jnp.float32),
        ],
    )
    def kern(t_hbm, seg_hbm, e_hbm, z_hbm, out_hbm, e_tab, t_buf, s_buf, e_buf, den_sh):
        c = lax.axis_index("c")
        s = lax.axis_index("s")
        wid = s * NC + c
        pltpu.sync_copy(e_hbm, e_tab)

        @pl.when(s == 0)
        def _():
            pltpu.sync_copy(z_hbm, den_sh)

        plsc.subcore_barrier()
        base = wid * per_w

        def body(g, carry):
            off = base + g * K
            pltpu.sync_copy(t_hbm.at[pl.ds(off, K)], t_buf)
            pltpu.sync_copy(seg_hbm.at[pl.ds(off, K)], s_buf.at[0])
            for j in range(K // L):
                sl = pl.ds(j * L, L)
                e_buf[0, sl] = plsc.load_gather(e_tab, [t_buf[sl]])
            pltpu.sync_copy(e_buf.at[0], den_sh.at[s_buf.at[0]], add=True)
            return carry

        lax.fori_loop(0, nch, body, 0)
        plsc.subcore_barrier()

        @pl.when(s == 0)
        def _():
            pltpu.sync_copy(den_sh, out_hbm.at[c])

    return kern(t_idx, seg_ids, e_all, zeros_p)


def _pool_sc(protein_h, t_idx, seg_ids, e_all, denom, ptd, zeros_nd, zeros_n, n_drugs):
    m = t_idx.shape[0]
    d = protein_h.shape[1]
    per_w = m // NW
    nch = per_w // K
    n_pathways = ptd.shape[0]
    mesh = plsc.VectorSubcoreMesh(core_axis_name="c", subcore_axis_name="s",
                                  num_cores=NC, num_subcores=NS)

    @functools.partial(
        pl.kernel,
        out_type=(
            jax.ShapeDtypeStruct((NC, n_drugs, d), jnp.float32),
            jax.ShapeDtypeStruct((NC, n_drugs), jnp.float32),
        ),
        mesh=mesh,
        compiler_params=pltpu.CompilerParams(needs_layout_passes=False),
        scratch_types=(
            [
                pltpu.VMEM((3, K), jnp.int32),
                pltpu.VMEM((3, K), jnp.int32),
                pltpu.VMEM((3, K), jnp.float32),
                pltpu.VMEM((3, K), jnp.float32),
                pltpu.VMEM((3, K), jnp.int32),
                pltpu.VMEM((3, K), jnp.float32),
                pltpu.VMEM((3, K, d), jnp.float32),
            ]
            + [pltpu.SemaphoreType.DMA] * 18
            + [
                pltpu.VMEM_SHARED((n_drugs, d), jnp.float32),
                pltpu.VMEM_SHARED((n_drugs,), jnp.float32),
            ]
        ),
    )
    def kern(ph_hbm, t_hbm, seg_hbm, e_hbm, den_hbm, ptd_hbm, znd_hbm, zn_hbm,
             dsum_hbm, cnt_hbm, *scr):
        t_buf, s_buf, e_buf, dn_buf, dg_buf, cf_buf, row_buf = scr[:7]
        semr = scr[7:10]
        seme = scr[10:13]
        semd = scr[13:16]
        semp = scr[16:19]
        semw = scr[19:22]
        semc = scr[22:25]
        dsum_sh, cnt_sh = scr[25], scr[26]
        c = lax.axis_index("c")
        s = lax.axis_index("s")
        wid = s * NC + c

        @pl.when(s == 0)
        def _():
            pltpu.sync_copy(znd_hbm, dsum_sh)
            pltpu.sync_copy(zn_hbm, cnt_sh)

        plsc.subcore_barrier()
        base = wid * per_w

        def wait_scatter(b):
            pltpu.make_async_copy(row_buf.at[b], dsum_sh.at[dg_buf.at[b]],
                                  semw[b]).wait()
            pltpu.make_async_copy(cf_buf.at[b], cnt_sh.at[dg_buf.at[b]],
                                  semc[b]).wait()

        def issue(g, b, first=False):
            if not first:
                wait_scatter(b)
            off = base + g * K
            pltpu.sync_copy(t_hbm.at[pl.ds(off, K)], t_buf.at[b])
            pltpu.sync_copy(seg_hbm.at[pl.ds(off, K)], s_buf.at[b])
            pltpu.async_copy(ph_hbm.at[t_buf.at[b]], row_buf.at[b], semr[b])
            pltpu.async_copy(e_hbm.at[t_buf.at[b]], e_buf.at[b], seme[b])
            pltpu.async_copy(den_hbm.at[s_buf.at[b]], dn_buf.at[b], semd[b])
            pltpu.async_copy(ptd_hbm.at[s_buf.at[b]], dg_buf.at[b], semp[b])

        def process(b):
            pltpu.make_async_copy(e_hbm.at[t_buf.at[b]], e_buf.at[b],
                                  seme[b]).wait()
            pltpu.make_async_copy(den_hbm.at[s_buf.at[b]], dn_buf.at[b],
                                  semd[b]).wait()
            pltpu.make_async_copy(ptd_hbm.at[s_buf.at[b]], dg_buf.at[b],
                                  semp[b]).wait()
            for j in range(K // L):
                sl = pl.ds(j * L, L)
                cf_buf[b, sl] = e_buf[b, sl] / dn_buf[b, sl]
            pltpu.make_async_copy(ph_hbm.at[t_buf.at[b]], row_buf.at[b],
                                  semr[b]).wait()

            def rbody(i, rc):
                for u in range(2):
                    r = 2 * i + u
                    csp = plsc.load_gather(
                        cf_buf,
                        [jnp.full((L,), b, jnp.int32),
                         jnp.full((L,), r, jnp.int32)])
                    for q in range(d // L):
                        sl2 = pl.ds(q * L, L)
                        row_buf[b, r, sl2] = row_buf[b, r, sl2] * csp
                return rc

            lax.fori_loop(0, K // 2, rbody, 0)
            pltpu.make_async_copy(row_buf.at[b], dsum_sh.at[dg_buf.at[b]],
                                  semw[b]).start(add=True)
            pltpu.make_async_copy(cf_buf.at[b], cnt_sh.at[dg_buf.at[b]],
                                  semc[b]).start(add=True)

        def triple(i, first=False):
            issue(3 * i + 2, 2, first=first)
            process(0)
            issue(3 * i + 3, 0)
            process(1)
            issue(3 * i + 4, 1)
            process(2)

        issue(0, 0, first=True)
        issue(1, 1, first=True)
        triple(0, first=True)
        lax.fori_loop(1, nch // 3, lambda i, cy: (triple(i), cy)[1], 0)
        process(0)
        process(1)
        for u in range(3):
            wait_scatter(u)
        plsc.subcore_barrier()

        @pl.when(s == 0)
        def _():
            pltpu.sync_copy(dsum_sh, dsum_hbm.at[c])
            pltpu.sync_copy(cnt_sh, cnt_hbm.at[c])

    return kern(protein_h, t_idx, seg_ids, e_all, denom, ptd, zeros_nd, zeros_n)


def _finish_tc(dsum2, cnt2, proj_wT, proj_b):
    n, d = dsum2.shape[1], dsum2.shape[2]

    def body(ds_ref, ct_ref, pw_ref, pb_ref, out_ref):
        tot = ds_ref[0] + ds_ref[1]
        cnt = ct_ref[0] + ct_ref[1]
        avg = tot / jnp.maximum(cnt, 1.0)[:, None]
        r = jnp.dot(avg, pw_ref[...], preferred_element_type=jnp.float32)
        out_ref[...] = jnp.maximum(r + pb_ref[...], 0.0)

    return pl.pallas_call(
        body,
        out_shape=jax.ShapeDtypeStruct((n, d), jnp.float32),
    )(dsum2, cnt2, proj_wT, proj_b)


def kernel(protein_h, attn_w, proj_w, proj_b, drug_idx, protein_indices,
           pathway_segment_ids, pathway_to_drug):
    n_drugs = drug_idx.shape[0]
    n_pathways = pathway_to_drug.shape[0]
    d = protein_h.shape[1]

    e_all = _escore_tc(protein_h, attn_w)[:, 0]
    zeros_p = jnp.zeros((n_pathways,), jnp.float32)
    den2 = _denom_sc(protein_indices, pathway_segment_ids, e_all, zeros_p,
                     n_pathways)
    denom = den2[0] + den2[1]
    zeros_nd = jnp.zeros((n_drugs, d), jnp.float32)
    zeros_n = jnp.zeros((n_drugs,), jnp.float32)
    dsum2, cnt2 = _pool_sc(protein_h, protein_indices, pathway_segment_ids,
                           e_all, denom, pathway_to_drug, zeros_nd, zeros_n,
                           n_drugs)
    return _finish_tc(dsum2, cnt2, proj_w.T, jnp.reshape(proj_b, (1, d)))


# speedup vs baseline: 16.8949x; 1.3151x over previous
import functools

import jax
import jax.numpy as jnp
from jax import lax
from jax.experimental import pallas as pl
from jax.experimental.pallas import tpu as pltpu
from jax.experimental.pallas import tpu_sc as plsc

NC = 2
NS = 16
NW = NC * NS
L = 16
K = 80


def _escore_tc(protein_h, attn_w):
    def body(ph_ref, aw_ref, out_ref):
        s = jnp.dot(ph_ref[...], aw_ref[...], preferred_element_type=jnp.float32)
        out_ref[...] = jnp.exp(s - jnp.max(s))
    return pl.pallas_call(
        body,
        out_shape=jax.ShapeDtypeStruct((protein_h.shape[0], 1), jnp.float32),
    )(protein_h, attn_w)


def _denom_sc(t_idx, seg_ids, e_all, zeros_p, n_pathways):
    m = t_idx.shape[0]
    per_w = m // NW
    nch = per_w // K
    mesh = plsc.VectorSubcoreMesh(core_axis_name="c", subcore_axis_name="s",
                                  num_cores=NC, num_subcores=NS)

    @functools.partial(
        pl.kernel,
        out_type=jax.ShapeDtypeStruct((NC, n_pathways), jnp.float32),
        mesh=mesh,
        compiler_params=pltpu.CompilerParams(needs_layout_passes=False),
        scratch_types=[
            pltpu.VMEM((e_all.shape[0],), jnp.float32),
            pltpu.VMEM((K,), jnp.int32),
            pltpu.VMEM((1, K), jnp.int32),
            pltpu.VMEM((1, K), jnp.float32),
            pltpu.VMEM_SHARED((n_pathways,), jnp.float32),
        ],
    )
    def kern(t_hbm, seg_hbm, e_hbm, z_hbm, out_hbm, e_tab, t_buf, s_buf, e_buf, den_sh):
        c = lax.axis_index("c")
        s = lax.axis_index("s")
        wid = s * NC + c
        pltpu.sync_copy(e_hbm, e_tab)

        @pl.when(s == 0)
        def _():
            pltpu.sync_copy(z_hbm, den_sh)

        plsc.subcore_barrier()
        base = wid * per_w

        def body(g, carry):
            off = base + g * K
            pltpu.sync_copy(t_hbm.at[pl.ds(off, K)], t_buf)
            pltpu.sync_copy(seg_hbm.at[pl.ds(off, K)], s_buf.at[0])
            for j in range(K // L):
                sl = pl.ds(j * L, L)
                e_buf[0, sl] = plsc.load_gather(e_tab, [t_buf[sl]])
            pltpu.sync_copy(e_buf.at[0], den_sh.at[s_buf.at[0]], add=True)
            return carry

        lax.fori_loop(0, nch, body, 0)
        plsc.subcore_barrier()

        @pl.when(s == 0)
        def _():
            pltpu.sync_copy(den_sh, out_hbm.at[c])

    return kern(t_idx, seg_ids, e_all, zeros_p)


def _pool_sc(protein_h, t_idx, seg_ids, e_all, denom, ptd, zeros_nd, zeros_n, n_drugs):
    m = t_idx.shape[0]
    d = protein_h.shape[1]
    per_w = m // NW
    nch = per_w // K
    n_pathways = ptd.shape[0]
    mesh = plsc.VectorSubcoreMesh(core_axis_name="c", subcore_axis_name="s",
                                  num_cores=NC, num_subcores=NS)

    @functools.partial(
        pl.kernel,
        out_type=(
            jax.ShapeDtypeStruct((NC, n_drugs, d), jnp.float32),
            jax.ShapeDtypeStruct((NC, n_drugs), jnp.float32),
        ),
        mesh=mesh,
        compiler_params=pltpu.CompilerParams(needs_layout_passes=False),
        scratch_types=(
            [
                pltpu.VMEM((3, K), jnp.int32),
                pltpu.VMEM((3, K), jnp.int32),
                pltpu.VMEM((3, K), jnp.float32),
                pltpu.VMEM((3, K), jnp.float32),
                pltpu.VMEM((3, K), jnp.int32),
                pltpu.VMEM((3, K), jnp.float32),
                pltpu.VMEM((3, K, d), jnp.float32),
            ]
            + [pltpu.SemaphoreType.DMA] * 24
            + [
                pltpu.VMEM_SHARED((n_drugs, d), jnp.float32),
                pltpu.VMEM_SHARED((n_drugs,), jnp.float32),
            ]
        ),
    )
    def kern(ph_hbm, t_hbm, seg_hbm, e_hbm, den_hbm, ptd_hbm, znd_hbm, zn_hbm,
             dsum_hbm, cnt_hbm, *scr):
        t_buf, s_buf, e_buf, dn_buf, dg_buf, cf_buf, row_buf = scr[:7]
        semr = scr[7:10]
        seme = scr[10:13]
        semd = scr[13:16]
        semp = scr[16:19]
        semw = scr[19:22]
        semc = scr[22:25]
        semt = scr[25:28]
        semg = scr[28:31]
        dsum_sh, cnt_sh = scr[31], scr[32]
        c = lax.axis_index("c")
        s = lax.axis_index("s")
        wid = s * NC + c

        @pl.when(s == 0)
        def _():
            pltpu.sync_copy(znd_hbm, dsum_sh)
            pltpu.sync_copy(zn_hbm, cnt_sh)

        plsc.subcore_barrier()
        base = wid * per_w

        def wait_scatter(b):
            pltpu.make_async_copy(row_buf.at[b], dsum_sh.at[dg_buf.at[b]],
                                  semw[b]).wait()
            pltpu.make_async_copy(cf_buf.at[b], cnt_sh.at[dg_buf.at[b]],
                                  semc[b]).wait()

        def issue_idx(g, b):
            off = base + g * K
            pltpu.async_copy(t_hbm.at[pl.ds(off, K)], t_buf.at[b], semt[b])
            pltpu.async_copy(seg_hbm.at[pl.ds(off, K)], s_buf.at[b], semg[b])

        def issue_gathers(g, b, first=False):
            if not first:
                wait_scatter(b)
            off = base + g * K
            pltpu.make_async_copy(t_hbm.at[pl.ds(off, K)], t_buf.at[b],
                                  semt[b]).wait()
            pltpu.make_async_copy(seg_hbm.at[pl.ds(off, K)], s_buf.at[b],
                                  semg[b]).wait()
            pltpu.async_copy(ph_hbm.at[t_buf.at[b]], row_buf.at[b], semr[b])
            pltpu.async_copy(e_hbm.at[t_buf.at[b]], e_buf.at[b], seme[b])
            pltpu.async_copy(den_hbm.at[s_buf.at[b]], dn_buf.at[b], semd[b])
            pltpu.async_copy(ptd_hbm.at[s_buf.at[b]], dg_buf.at[b], semp[b])

        def process(b):
            pltpu.make_async_copy(e_hbm.at[t_buf.at[b]], e_buf.at[b],
                                  seme[b]).wait()
            pltpu.make_async_copy(den_hbm.at[s_buf.at[b]], dn_buf.at[b],
                                  semd[b]).wait()
            pltpu.make_async_copy(ptd_hbm.at[s_buf.at[b]], dg_buf.at[b],
                                  semp[b]).wait()
            for j in range(K // L):
                sl = pl.ds(j * L, L)
                cf_buf[b, sl] = e_buf[b, sl] / dn_buf[b, sl]
            pltpu.make_async_copy(ph_hbm.at[t_buf.at[b]], row_buf.at[b],
                                  semr[b]).wait()

            def rbody(i, rc):
                for u in range(2):
                    r = 2 * i + u
                    csp = plsc.load_gather(
                        cf_buf,
                        [jnp.full((L,), b, jnp.int32),
                         jnp.full((L,), r, jnp.int32)])
                    for q in range(d // L):
                        sl2 = pl.ds(q * L, L)
                        row_buf[b, r, sl2] = row_buf[b, r, sl2] * csp
                return rc

            lax.fori_loop(0, K // 2, rbody, 0)
            pltpu.make_async_copy(row_buf.at[b], dsum_sh.at[dg_buf.at[b]],
                                  semw[b]).start(add=True)
            pltpu.make_async_copy(cf_buf.at[b], cnt_sh.at[dg_buf.at[b]],
                                  semc[b]).start(add=True)

        def triple(i, first=False):
            issue_gathers(3 * i + 1, 1, first=first)
            issue_idx(3 * i + 2, 2)
            process(0)
            issue_gathers(3 * i + 2, 2, first=first)
            issue_idx(3 * i + 3, 0)
            process(1)
            issue_gathers(3 * i + 3, 0)
            issue_idx(3 * i + 4, 1)
            process(2)

        issue_idx(0, 0)
        issue_idx(1, 1)
        issue_gathers(0, 0, first=True)
        triple(0, first=True)
        lax.fori_loop(1, nch // 3, lambda i, cy: (triple(i), cy)[1], 0)
        issue_gathers(nch - 1, (nch - 1) % 3)
        process((nch - 2) % 3)
        process((nch - 1) % 3)
        for u in range(3):
            wait_scatter(u)
        plsc.subcore_barrier()

        @pl.when(s == 0)
        def _():
            pltpu.sync_copy(dsum_sh, dsum_hbm.at[c])
            pltpu.sync_copy(cnt_sh, cnt_hbm.at[c])

    return kern(protein_h, t_idx, seg_ids, e_all, denom, ptd, zeros_nd, zeros_n)


def _finish_tc(dsum2, cnt2, proj_wT, proj_b):
    n, d = dsum2.shape[1], dsum2.shape[2]

    def body(ds_ref, ct_ref, pw_ref, pb_ref, out_ref):
        tot = ds_ref[0] + ds_ref[1]
        cnt = ct_ref[0] + ct_ref[1]
        avg = tot / jnp.maximum(cnt, 1.0)[:, None]
        r = jnp.dot(avg, pw_ref[...], preferred_element_type=jnp.float32)
        out_ref[...] = jnp.maximum(r + pb_ref[...], 0.0)

    return pl.pallas_call(
        body,
        out_shape=jax.ShapeDtypeStruct((n, d), jnp.float32),
    )(dsum2, cnt2, proj_wT, proj_b)


def kernel(protein_h, attn_w, proj_w, proj_b, drug_idx, protein_indices,
           pathway_segment_ids, pathway_to_drug):
    n_drugs = drug_idx.shape[0]
    n_pathways = pathway_to_drug.shape[0]
    d = protein_h.shape[1]

    e_all = _escore_tc(protein_h, attn_w)[:, 0]
    zeros_p = jnp.zeros((n_pathways,), jnp.float32)
    den2 = _denom_sc(protein_indices, pathway_segment_ids, e_all, zeros_p,
                     n_pathways)
    denom = den2[0] + den2[1]
    zeros_nd = jnp.zeros((n_drugs, d), jnp.float32)
    zeros_n = jnp.zeros((n_drugs,), jnp.float32)
    dsum2, cnt2 = _pool_sc(protein_h, protein_indices, pathway_segment_ids,
                           e_all, denom, pathway_to_drug, zeros_nd, zeros_n,
                           n_drugs)
    return _finish_tc(dsum2, cnt2, proj_w.T, jnp.reshape(proj_b, (1, d)))


# speedup vs baseline: 22.6007x; 1.3377x over previous
import functools

import jax
import jax.numpy as jnp
from jax import lax
from jax.experimental import pallas as pl
from jax.experimental.pallas import tpu as pltpu
from jax.experimental.pallas import tpu_sc as plsc

NC = 2
NS = 16
NW = NC * NS
L = 16
K = 80


def _escore_tc(protein_h, attn_w):
    def body(ph_ref, aw_ref, out_ref):
        s = jnp.dot(ph_ref[...], aw_ref[...], preferred_element_type=jnp.float32)
        out_ref[...] = jnp.exp(s - jnp.max(s))
    return pl.pallas_call(
        body,
        out_shape=jax.ShapeDtypeStruct((protein_h.shape[0], 1), jnp.float32),
    )(protein_h, attn_w)


def _denom_sc(t_idx, seg_ids, e_all, zeros_p, n_pathways):
    m = t_idx.shape[0]
    per_w = m // NW
    nch = per_w // K
    mesh = plsc.VectorSubcoreMesh(core_axis_name="c", subcore_axis_name="s",
                                  num_cores=NC, num_subcores=NS)

    @functools.partial(
        pl.kernel,
        out_type=jax.ShapeDtypeStruct((NC, n_pathways), jnp.float32),
        mesh=mesh,
        compiler_params=pltpu.CompilerParams(needs_layout_passes=False),
        scratch_types=(
            [
                pltpu.VMEM((e_all.shape[0],), jnp.float32),
                pltpu.VMEM((3, K), jnp.int32),
                pltpu.VMEM((3, K), jnp.int32),
                pltpu.VMEM((3, K), jnp.float32),
            ]
            + [pltpu.SemaphoreType.DMA] * 9
            + [pltpu.VMEM_SHARED((n_pathways,), jnp.float32)]
        ),
    )
    def kern(t_hbm, seg_hbm, e_hbm, z_hbm, out_hbm, *scr):
        e_tab, t_buf, s_buf, e_buf = scr[:4]
        semt = scr[4:7]
        semg = scr[7:10]
        semw = scr[10:13]
        den_sh = scr[13]
        c = lax.axis_index("c")
        s = lax.axis_index("s")
        wid = s * NC + c
        pltpu.sync_copy(e_hbm, e_tab)

        @pl.when(s == 0)
        def _():
            pltpu.sync_copy(z_hbm, den_sh)

        plsc.subcore_barrier()
        base = wid * per_w

        def issue_idx(g, b):
            off = base + g * K
            pltpu.async_copy(t_hbm.at[pl.ds(off, K)], t_buf.at[b], semt[b])
            pltpu.async_copy(seg_hbm.at[pl.ds(off, K)], s_buf.at[b], semg[b])

        def wait_scatter(b):
            pltpu.make_async_copy(e_buf.at[b], den_sh.at[s_buf.at[b]],
                                  semw[b]).wait()

        def process(g, b):
            off = base + g * K
            pltpu.make_async_copy(t_hbm.at[pl.ds(off, K)], t_buf.at[b],
                                  semt[b]).wait()
            pltpu.make_async_copy(seg_hbm.at[pl.ds(off, K)], s_buf.at[b],
                                  semg[b]).wait()
            for j in range(K // L):
                sl = pl.ds(j * L, L)
                e_buf[b, sl] = plsc.load_gather(e_tab, [t_buf[b, sl]])
            pltpu.make_async_copy(e_buf.at[b], den_sh.at[s_buf.at[b]],
                                  semw[b]).start(add=True)

        issue_idx(0, 0)
        issue_idx(1, 1)
        process(0, 0)
        issue_idx(2, 2)
        process(1, 1)
        wait_scatter(0)
        issue_idx(3, 0)
        process(2, 2)
        wait_scatter(1)
        issue_idx(4, 1)

        def triple(i, carry):
            g0 = 3 * i
            process(g0, 0)
            wait_scatter(2)
            issue_idx(g0 + 2, 2)
            process(g0 + 1, 1)
            wait_scatter(0)
            issue_idx(g0 + 3, 0)
            process(g0 + 2, 2)
            wait_scatter(1)
            issue_idx(g0 + 4, 1)
            return carry

        lax.fori_loop(1, nch // 3, triple, 0)
        process(nch - 2, (nch - 2) % 3)
        process(nch - 1, (nch - 1) % 3)
        for u in range(3):
            wait_scatter(u)
        plsc.subcore_barrier()

        @pl.when(s == 0)
        def _():
            pltpu.sync_copy(den_sh, out_hbm.at[c])

    return kern(t_idx, seg_ids, e_all, zeros_p)


def _pool_sc(protein_h, t_idx, seg_ids, e_all, denom, ptd, zeros_nd, zeros_n, n_drugs):
    m = t_idx.shape[0]
    d = protein_h.shape[1]
    per_w = m // NW
    nch = per_w // K
    n_pathways = ptd.shape[0]
    mesh = plsc.VectorSubcoreMesh(core_axis_name="c", subcore_axis_name="s",
                                  num_cores=NC, num_subcores=NS)

    @functools.partial(
        pl.kernel,
        out_type=(
            jax.ShapeDtypeStruct((NC, n_drugs, d), jnp.float32),
            jax.ShapeDtypeStruct((NC, n_drugs), jnp.float32),
        ),
        mesh=mesh,
        compiler_params=pltpu.CompilerParams(needs_layout_passes=False),
        scratch_types=(
            [
                pltpu.VMEM((3, K), jnp.int32),
                pltpu.VMEM((3, K), jnp.int32),
                pltpu.VMEM((3, K), jnp.float32),
                pltpu.VMEM((3, K), jnp.float32),
                pltpu.VMEM((3, K), jnp.int32),
                pltpu.VMEM((3, K), jnp.float32),
                pltpu.VMEM((3, K, d), jnp.float32),
            ]
            + [pltpu.SemaphoreType.DMA] * 24
            + [
                pltpu.VMEM_SHARED((n_drugs, d), jnp.float32),
                pltpu.VMEM_SHARED((n_drugs,), jnp.float32),
            ]
        ),
    )
    def kern(ph_hbm, t_hbm, seg_hbm, e_hbm, den_hbm, ptd_hbm, znd_hbm, zn_hbm,
             dsum_hbm, cnt_hbm, *scr):
        t_buf, s_buf, e_buf, dn_buf, dg_buf, cf_buf, row_buf = scr[:7]
        semr = scr[7:10]
        seme = scr[10:13]
        semd = scr[13:16]
        semp = scr[16:19]
        semw = scr[19:22]
        semc = scr[22:25]
        semt = scr[25:28]
        semg = scr[28:31]
        dsum_sh, cnt_sh = scr[31], scr[32]
        c = lax.axis_index("c")
        s = lax.axis_index("s")
        wid = s * NC + c

        @pl.when(s == 0)
        def _():
            pltpu.sync_copy(znd_hbm, dsum_sh)
            pltpu.sync_copy(zn_hbm, cnt_sh)

        plsc.subcore_barrier()
        base = wid * per_w

        def wait_scatter(b):
            pltpu.make_async_copy(row_buf.at[b], dsum_sh.at[dg_buf.at[b]],
                                  semw[b]).wait()
            pltpu.make_async_copy(cf_buf.at[b], cnt_sh.at[dg_buf.at[b]],
                                  semc[b]).wait()

        def issue_idx(g, b):
            off = base + g * K
            pltpu.async_copy(t_hbm.at[pl.ds(off, K)], t_buf.at[b], semt[b])
            pltpu.async_copy(seg_hbm.at[pl.ds(off, K)], s_buf.at[b], semg[b])

        def issue_gathers(g, b, first=False):
            if not first:
                wait_scatter(b)
            off = base + g * K
            pltpu.make_async_copy(t_hbm.at[pl.ds(off, K)], t_buf.at[b],
                                  semt[b]).wait()
            pltpu.make_async_copy(seg_hbm.at[pl.ds(off, K)], s_buf.at[b],
                                  semg[b]).wait()
            pltpu.async_copy(ph_hbm.at[t_buf.at[b]], row_buf.at[b], semr[b])
            pltpu.async_copy(e_hbm.at[t_buf.at[b]], e_buf.at[b], seme[b])
            pltpu.async_copy(den_hbm.at[s_buf.at[b]], dn_buf.at[b], semd[b])
            pltpu.async_copy(ptd_hbm.at[s_buf.at[b]], dg_buf.at[b], semp[b])

        def process(b):
            pltpu.make_async_copy(e_hbm.at[t_buf.at[b]], e_buf.at[b],
                                  seme[b]).wait()
            pltpu.make_async_copy(den_hbm.at[s_buf.at[b]], dn_buf.at[b],
                                  semd[b]).wait()
            pltpu.make_async_copy(ptd_hbm.at[s_buf.at[b]], dg_buf.at[b],
                                  semp[b]).wait()
            for j in range(K // L):
                sl = pl.ds(j * L, L)
                cf_buf[b, sl] = e_buf[b, sl] / dn_buf[b, sl]
            pltpu.make_async_copy(ph_hbm.at[t_buf.at[b]], row_buf.at[b],
                                  semr[b]).wait()

            def rbody(i, rc):
                for u in range(2):
                    r = 2 * i + u
                    csp = plsc.load_gather(
                        cf_buf,
                        [jnp.full((L,), b, jnp.int32),
                         jnp.full((L,), r, jnp.int32)])
                    for q in range(d // L):
                        sl2 = pl.ds(q * L, L)
                        row_buf[b, r, sl2] = row_buf[b, r, sl2] * csp
                return rc

            lax.fori_loop(0, K // 2, rbody, 0)
            pltpu.make_async_copy(row_buf.at[b], dsum_sh.at[dg_buf.at[b]],
                                  semw[b]).start(add=True)
            pltpu.make_async_copy(cf_buf.at[b], cnt_sh.at[dg_buf.at[b]],
                                  semc[b]).start(add=True)

        def triple(i, first=False):
            issue_gathers(3 * i + 1, 1, first=first)
            issue_idx(3 * i + 2, 2)
            process(0)
            issue_gathers(3 * i + 2, 2, first=first)
            issue_idx(3 * i + 3, 0)
            process(1)
            issue_gathers(3 * i + 3, 0)
            issue_idx(3 * i + 4, 1)
            process(2)

        issue_idx(0, 0)
        issue_idx(1, 1)
        issue_gathers(0, 0, first=True)
        triple(0, first=True)
        lax.fori_loop(1, nch // 3, lambda i, cy: (triple(i), cy)[1], 0)
        issue_gathers(nch - 1, (nch - 1) % 3)
        process((nch - 2) % 3)
        process((nch - 1) % 3)
        for u in range(3):
            wait_scatter(u)
        plsc.subcore_barrier()

        @pl.when(s == 0)
        def _():
            pltpu.sync_copy(dsum_sh, dsum_hbm.at[c])
            pltpu.sync_copy(cnt_sh, cnt_hbm.at[c])

    return kern(protein_h, t_idx, seg_ids, e_all, denom, ptd, zeros_nd, zeros_n)


def _finish_tc(dsum2, cnt2, proj_wT, proj_b):
    n, d = dsum2.shape[1], dsum2.shape[2]

    def body(ds_ref, ct_ref, pw_ref, pb_ref, out_ref):
        tot = ds_ref[0] + ds_ref[1]
        cnt = ct_ref[0] + ct_ref[1]
        avg = tot / jnp.maximum(cnt, 1.0)[:, None]
        r = jnp.dot(avg, pw_ref[...], preferred_element_type=jnp.float32)
        out_ref[...] = jnp.maximum(r + pb_ref[...], 0.0)

    return pl.pallas_call(
        body,
        out_shape=jax.ShapeDtypeStruct((n, d), jnp.float32),
    )(dsum2, cnt2, proj_wT, proj_b)


def kernel(protein_h, attn_w, proj_w, proj_b, drug_idx, protein_indices,
           pathway_segment_ids, pathway_to_drug):
    n_drugs = drug_idx.shape[0]
    n_pathways = pathway_to_drug.shape[0]
    d = protein_h.shape[1]

    e_all = _escore_tc(protein_h, attn_w)[:, 0]
    zeros_p = jnp.zeros((n_pathways,), jnp.float32)
    den2 = _denom_sc(protein_indices, pathway_segment_ids, e_all, zeros_p,
                     n_pathways)
    denom = den2[0] + den2[1]
    zeros_nd = jnp.zeros((n_drugs, d), jnp.float32)
    zeros_n = jnp.zeros((n_drugs,), jnp.float32)
    dsum2, cnt2 = _pool_sc(protein_h, protein_indices, pathway_segment_ids,
                           e_all, denom, pathway_to_drug, zeros_nd, zeros_n,
                           n_drugs)
    return _finish_tc(dsum2, cnt2, proj_w.T, jnp.reshape(proj_b, (1, d)))
